# Initial kernel scaffold; baseline (speedup 1.0000x reference)
import jax
import jax.numpy as jnp
from jax.experimental import pallas as pl


def kernel(x, edge_index, W1, b1, W2, b2):
    raise NotImplementedError("write your pallas kernel here")


# speedup vs baseline: 19.0805x; 19.0805x over previous
import functools

import jax
import jax.numpy as jnp
from jax import lax
from jax.experimental import pallas as pl
from jax.experimental.pallas import tpu as pltpu
from jax.experimental.pallas import tpu_sc as plsc

N = 10000
E = 320000
D = 128

NC = 2
NS = 16
NW = NC * NS
EPT = E // NW
CHUNK = 80
NCHUNK = EPT // CHUNK
NPAD = 10240
RPT = NPAD // NS

_mesh = plsc.VectorSubcoreMesh(core_axis_name="c", subcore_axis_name="s",
                               num_cores=NC, num_subcores=NS)


def _zero_buf(buf, nrows, ncols):
    zeros16 = jnp.zeros((16,), jnp.float32)

    def body(i, _):
        for j in range(ncols // 16):
            buf[i, pl.ds(j * 16, 16)] = zeros16
        return 0

    lax.fori_loop(0, nrows, body, 0)


def _deg_body(dst_hbm, deg_hbm, deg_sp, dst_v, ones_v):
    c = lax.axis_index("c")
    s = lax.axis_index("s")
    wid = c * NS + s

    _zero_buf(ones_v, CHUNK, 16)
    for z in range(RPT // CHUNK):
        pltpu.sync_copy(ones_v, deg_sp.at[pl.ds(s * RPT + z * CHUNK, CHUNK)])

    ones16 = jnp.ones((16,), jnp.float32)

    def ones_body(i, _):
        ones_v[i, :] = ones16
        return 0

    lax.fori_loop(0, CHUNK, ones_body, 0)

    pltpu.sync_copy(dst_hbm.at[wid], dst_v)
    plsc.subcore_barrier()

    def chunk(k, _):
        pltpu.sync_copy(ones_v, deg_sp.at[dst_v.at[k]], add=True)
        return 0

    lax.fori_loop(0, NCHUNK, chunk, 0)
    plsc.subcore_barrier()

    pltpu.sync_copy(deg_sp.at[pl.ds(s * RPT, RPT)],
                    deg_hbm.at[c].at[pl.ds(s * RPT, RPT)])


_deg_call = pl.kernel(
    _deg_body,
    out_type=jax.ShapeDtypeStruct((NC, NPAD, 16), jnp.float32),
    mesh=_mesh,
    scratch_types=[
        pltpu.VMEM_SHARED((NPAD, 16), jnp.float32),
        pltpu.VMEM((NCHUNK, CHUNK), jnp.int32),
        pltpu.VMEM((CHUNK, 16), jnp.float32),
    ],
)


def _agg_body(g_hbm, src_hbm, dst_hbm, out_hbm, acc_sp, src_v, dst_v, rows_v,
              sem):
    c = lax.axis_index("c")
    s = lax.axis_index("s")
    wid = c * NS + s

    _zero_buf(rows_v, CHUNK, D)
    for z in range(RPT // CHUNK):
        pltpu.sync_copy(rows_v, acc_sp.at[pl.ds(s * RPT + z * CHUNK, CHUNK)])

    pltpu.sync_copy(src_hbm.at[wid], src_v)
    pltpu.sync_copy(dst_hbm.at[wid], dst_v)
    plsc.subcore_barrier()

    def chunk(k, _):
        pltpu.async_copy(g_hbm.at[src_v.at[k]], rows_v, sem).wait()
        pltpu.sync_copy(rows_v, acc_sp.at[dst_v.at[k]], add=True)
        return 0

    lax.fori_loop(0, NCHUNK, chunk, 0)
    plsc.subcore_barrier()

    pltpu.sync_copy(acc_sp.at[pl.ds(s * RPT, RPT)],
                    out_hbm.at[c].at[pl.ds(s * RPT, RPT)])


_agg_call = pl.kernel(
    _agg_body,
    out_type=jax.ShapeDtypeStruct((NC, NPAD, D), jnp.float32),
    mesh=_mesh,
    scratch_types=[
        pltpu.VMEM_SHARED((NPAD, D), jnp.float32),
        pltpu.VMEM((NCHUNK, CHUNK), jnp.int32),
        pltpu.VMEM((NCHUNK, CHUNK), jnp.int32),
        pltpu.VMEM((CHUNK, D), jnp.float32),
        pltpu.SemaphoreType.DMA,
    ],
)


BR = 2000


def _dinv(dga_ref, dgb_ref):
    return lax.rsqrt(dga_ref[:, :1] + dgb_ref[:, :1] + 1.0)


def _k1_body(x_ref, w_ref, dga_ref, dgb_ref, g_ref):
    h = jnp.dot(x_ref[...], w_ref[...], preferred_element_type=jnp.float32)
    g_ref[...] = h * _dinv(dga_ref, dgb_ref)


def _k2_body(sa_ref, sb_ref, g1_ref, dga_ref, dgb_ref, b1_ref, w2_ref, g2_ref):
    dinv = _dinv(dga_ref, dgb_ref)
    y = (sa_ref[...] + sb_ref[...] + g1_ref[...]) * dinv + b1_ref[...]
    y = jnp.maximum(y, 0.0)
    g2_ref[...] = jnp.dot(y, w2_ref[...],
                          preferred_element_type=jnp.float32) * dinv


def _k3_body(sa_ref, sb_ref, g2_ref, dga_ref, dgb_ref, b2_ref, o_ref):
    o_ref[...] = ((sa_ref[...] + sb_ref[...] + g2_ref[...])
                  * _dinv(dga_ref, dgb_ref) + b2_ref[...])


def _row_spec(w):
    return pl.BlockSpec((BR, w), lambda i: (i, 0))


_full_mat = pl.BlockSpec((D, D), lambda i: (0, 0))
_full_vec = pl.BlockSpec((1, D), lambda i: (0, 0))

_k1_call = pl.pallas_call(
    _k1_body,
    grid=(N // BR,),
    in_specs=[_row_spec(D), _full_mat, _row_spec(16), _row_spec(16)],
    out_specs=_row_spec(D),
    out_shape=jax.ShapeDtypeStruct((N, D), jnp.float32),
)

_k2_call = pl.pallas_call(
    _k2_body,
    grid=(N // BR,),
    in_specs=[_row_spec(D), _row_spec(D), _row_spec(D), _row_spec(16),
              _row_spec(16), _full_vec, _full_mat],
    out_specs=_row_spec(D),
    out_shape=jax.ShapeDtypeStruct((N, D), jnp.float32),
)

_k3_call = pl.pallas_call(
    _k3_body,
    grid=(N // BR,),
    in_specs=[_row_spec(D), _row_spec(D), _row_spec(D), _row_spec(16),
              _row_spec(16), _full_vec],
    out_specs=_row_spec(D),
    out_shape=jax.ShapeDtypeStruct((N, D), jnp.float32),
)


def kernel(x, edge_index, W1, b1, W2, b2):
    src = edge_index[0].reshape(NW, NCHUNK, CHUNK)
    dst = edge_index[1].reshape(NW, NCHUNK, CHUNK)

    deg = _deg_call(dst)
    dga, dgb = deg[0, :N], deg[1, :N]

    g1 = _k1_call(x, W1, dga, dgb)
    s1 = _agg_call(g1, src, dst)
    g2 = _k2_call(s1[0, :N], s1[1, :N], g1, dga, dgb, b1.reshape(1, D), W2)
    s2 = _agg_call(g2, src, dst)
    return _k3_call(s2[0, :N], s2[1, :N], g2, dga, dgb, b2.reshape(1, D))
